# Initial kernel scaffold; baseline (speedup 1.0000x reference)
import jax
import jax.numpy as jnp
from jax.experimental import pallas as pl


def kernel(edge_attr, W0, W1, W2):
    raise NotImplementedError("write your pallas kernel here")


# speedup vs baseline: 1.6078x; 1.6078x over previous
import functools

import jax
import jax.numpy as jnp
from jax import lax
from jax.experimental import pallas as pl
from jax.experimental.pallas import tpu as pltpu
from jax.experimental.pallas import tpu_sc as plsc

D = 128
CPAD = 128
KPAD = 8


def _fused_table(w0p, w1p, w2p, n1, n2):
    def body(w0_ref, w1_ref, w2_ref, t_ref):
        ci = lax.broadcasted_iota(jnp.int32, (CPAD, KPAD), 0)
        j = lax.broadcasted_iota(jnp.int32, (CPAD, KPAD), 1)
        oh0 = (ci // (n1 * n2) == j).astype(jnp.float32)
        oh1 = ((ci // n2) % n1 == j).astype(jnp.float32)
        oh2 = (ci % n2 == j).astype(jnp.float32)
        t = jnp.dot(oh0, w0_ref[...], preferred_element_type=jnp.float32)
        t += jnp.dot(oh1, w1_ref[...], preferred_element_type=jnp.float32)
        t += jnp.dot(oh2, w2_ref[...], preferred_element_type=jnp.float32)
        t_ref[...] = t

    return pl.pallas_call(
        body,
        out_shape=jax.ShapeDtypeStruct((CPAD, D), jnp.float32),
    )(w0p, w1p, w2p)


def _combined_index(edge_attr, n1, n2):
    E = edge_attr.shape[0]
    BE = 512
    assert E % BE == 0

    def body(ea_ref, idx_ref):
        ea = ea_ref[...]
        idx_ref[...] = ea[:, 0] * (n1 * n2) + ea[:, 1] * n2 + ea[:, 2]

    return pl.pallas_call(
        body,
        grid=(E // BE,),
        in_specs=[pl.BlockSpec((BE, 3), lambda i: (i, 0))],
        out_specs=pl.BlockSpec((BE,), lambda i: (i,)),
        out_shape=jax.ShapeDtypeStruct((E,), jnp.int32),
    )(edge_attr)


def _sc_lookup(table, idx, E):
    info = plsc.get_sparse_core_info()
    NC, NS = info.num_cores, info.num_subcores
    NW = NC * NS
    per_w = E // NW
    CH = 80
    n_ch = per_w // CH
    assert per_w % CH == 0 and E % NW == 0

    mesh = plsc.VectorSubcoreMesh(core_axis_name="c", subcore_axis_name="s")

    @functools.partial(
        pl.kernel,
        mesh=mesh,
        out_type=jax.ShapeDtypeStruct((E, D), jnp.float32),
        scratch_types=[
            pltpu.VMEM((per_w,), jnp.int32),
            pltpu.VMEM((CH, D), jnp.float32),
            pltpu.SemaphoreType.DMA,
        ],
    )
    def k(t_hbm, idx_hbm, out_hbm, idx_v, rows_v, sem):
        wid = lax.axis_index("s") * NC + lax.axis_index("c")
        base = wid * per_w
        pltpu.sync_copy(idx_hbm.at[pl.ds(base, per_w)], idx_v)

        def ch_body(kk, carry):
            e0 = base + kk * CH
            pltpu.async_copy(
                t_hbm.at[idx_v.at[pl.ds(kk * CH, CH)]], rows_v, sem
            ).wait()
            pltpu.sync_copy(rows_v, out_hbm.at[pl.ds(e0, CH)])
            return carry

        lax.fori_loop(0, n_ch, ch_body, 0)

    return k(table, idx)


def kernel(edge_attr, W0, W1, W2):
    E = edge_attr.shape[0]
    n0, n1, n2 = W0.shape[0], W1.shape[0], W2.shape[0]
    assert n0 * n1 * n2 <= CPAD and max(n0, n1, n2) <= KPAD

    def pad(w):
        return jnp.pad(w, ((0, KPAD - w.shape[0]), (0, 0)))

    table = _fused_table(pad(W0), pad(W1), pad(W2), n1, n2)
    idx = _combined_index(edge_attr, n1, n2)
    return _sc_lookup(table, idx, E)


# speedup vs baseline: 1.6364x; 1.0178x over previous
import functools

import jax
import jax.numpy as jnp
from jax import lax
from jax.experimental import pallas as pl
from jax.experimental.pallas import tpu as pltpu
from jax.experimental.pallas import tpu_sc as plsc

D = 128
CPAD = 128
KPAD = 8


def _fused_table(w0p, w1p, w2p, n1, n2):
    def body(w0_ref, w1_ref, w2_ref, t_ref):
        ci = lax.broadcasted_iota(jnp.int32, (CPAD, KPAD), 0)
        j = lax.broadcasted_iota(jnp.int32, (CPAD, KPAD), 1)
        oh0 = (ci // (n1 * n2) == j).astype(jnp.float32)
        oh1 = ((ci // n2) % n1 == j).astype(jnp.float32)
        oh2 = (ci % n2 == j).astype(jnp.float32)
        t = jnp.dot(oh0, w0_ref[...], preferred_element_type=jnp.float32)
        t += jnp.dot(oh1, w1_ref[...], preferred_element_type=jnp.float32)
        t += jnp.dot(oh2, w2_ref[...], preferred_element_type=jnp.float32)
        t_ref[...] = t

    return pl.pallas_call(
        body,
        out_shape=jax.ShapeDtypeStruct((CPAD, D), jnp.float32),
    )(w0p, w1p, w2p)


def _combined_index(edge_attr, n1, n2):
    E = edge_attr.shape[0]
    BE = 512
    assert E % BE == 0

    def body(ea_ref, idx_ref):
        ea = ea_ref[...]
        idx_ref[...] = ea[:, 0] * (n1 * n2) + ea[:, 1] * n2 + ea[:, 2]

    return pl.pallas_call(
        body,
        grid=(E // BE,),
        in_specs=[pl.BlockSpec((BE, 3), lambda i: (i, 0))],
        out_specs=pl.BlockSpec((BE,), lambda i: (i,)),
        out_shape=jax.ShapeDtypeStruct((E,), jnp.int32),
    )(edge_attr)


def _sc_lookup(table, idx, E):
    info = plsc.get_sparse_core_info()
    NC, NS = info.num_cores, info.num_subcores
    NW = NC * NS
    per_w = E // NW
    CH = 80
    n_ch = per_w // CH
    assert per_w % CH == 0 and E % NW == 0

    NBUF = 5
    LAG = 2
    assert n_ch % NBUF == 0 and n_ch >= 2 * NBUF

    mesh = plsc.VectorSubcoreMesh(core_axis_name="c", subcore_axis_name="s")

    @functools.partial(
        pl.kernel,
        mesh=mesh,
        out_type=jax.ShapeDtypeStruct((E, D), jnp.float32),
        scratch_types=[
            pltpu.VMEM((per_w,), jnp.int32),
            pltpu.VMEM((NBUF * CH, D), jnp.float32),
            *([pltpu.SemaphoreType.DMA] * (2 * NBUF)),
        ],
    )
    def k(t_hbm, idx_hbm, out_hbm, idx_v, rows_v, *sems):
        sem_g, sem_s = sems[:NBUF], sems[NBUF:]
        wid = lax.axis_index("s") * NC + lax.axis_index("c")
        base = wid * per_w
        pltpu.sync_copy(idx_hbm.at[pl.ds(base, per_w)], idx_v)

        def buf(b):
            return rows_v.at[pl.ds(b * CH, CH)]

        def start_gather(kk, b):
            pltpu.async_copy(t_hbm.at[idx_v.at[pl.ds(kk * CH, CH)]],
                             buf(b), sem_g[b])

        def start_scatter(kk, b):
            pltpu.async_copy(buf(b), out_hbm.at[pl.ds(base + kk * CH, CH)],
                             sem_s[b])

        def wait_gather(b):
            pltpu.make_async_copy(out_hbm.at[pl.ds(base, CH)], buf(b),
                                  sem_g[b]).wait()

        def wait_scatter(b):
            pltpu.make_async_copy(buf(b), out_hbm.at[pl.ds(base, CH)],
                                  sem_s[b]).wait()

        for kb in range(NBUF):
            start_gather(kb, kb)
            if kb >= LAG:
                wait_gather(kb - LAG)
                start_scatter(kb - LAG, kb - LAG)

        def outer(g, carry):
            k0 = NBUF + g * NBUF
            for b in range(NBUF):
                kk = k0 + b
                wait_scatter(b)
                start_gather(kk, b)
                bs = (b - LAG) % NBUF
                wait_gather(bs)
                start_scatter(kk - LAG, bs)
            return carry

        lax.fori_loop(0, (n_ch - NBUF) // NBUF, outer, 0)

        for i in range(LAG):
            kk = n_ch - LAG + i
            wait_gather(kk % NBUF)
            start_scatter(kk, kk % NBUF)
        for b in range(NBUF):
            wait_scatter(b)

    return k(table, idx)


def kernel(edge_attr, W0, W1, W2):
    E = edge_attr.shape[0]
    n0, n1, n2 = W0.shape[0], W1.shape[0], W2.shape[0]
    assert n0 * n1 * n2 <= CPAD and max(n0, n1, n2) <= KPAD

    def pad(w):
        return jnp.pad(w, ((0, KPAD - w.shape[0]), (0, 0)))

    table = _fused_table(pad(W0), pad(W1), pad(W2), n1, n2)
    idx = _combined_index(edge_attr, n1, n2)
    return _sc_lookup(table, idx, E)


# speedup vs baseline: 2.6478x; 1.6180x over previous
import functools

import jax
import jax.numpy as jnp
from jax import lax
from jax.experimental import pallas as pl
from jax.experimental.pallas import tpu as pltpu
from jax.experimental.pallas import tpu_sc as plsc

D = 128
CPAD = 128
KPAD = 8


def _fused_table(w0p, w1p, w2p, n1, n2, rep):
    def body(w0_ref, w1_ref, w2_ref, t_ref):
        ci = lax.broadcasted_iota(jnp.int32, (CPAD, KPAD), 0)
        j = lax.broadcasted_iota(jnp.int32, (CPAD, KPAD), 1)
        oh0 = (ci // (n1 * n2) == j).astype(jnp.float32)
        oh1 = ((ci // n2) % n1 == j).astype(jnp.float32)
        oh2 = (ci % n2 == j).astype(jnp.float32)
        t = jnp.dot(oh0, w0_ref[...], preferred_element_type=jnp.float32)
        t += jnp.dot(oh1, w1_ref[...], preferred_element_type=jnp.float32)
        t += jnp.dot(oh2, w2_ref[...], preferred_element_type=jnp.float32)
        t_ref[0] = t

    return pl.pallas_call(
        body,
        grid=(rep,),
        in_specs=[pl.BlockSpec((KPAD, D), lambda i: (0, 0))] * 3,
        out_specs=pl.BlockSpec((1, CPAD, D), lambda i: (i, 0, 0)),
        out_shape=jax.ShapeDtypeStruct((rep, CPAD, D), jnp.float32),
    )(w0p, w1p, w2p)


def _combined_index(edge_attr, n1, n2, per_w):
    E = edge_attr.shape[0]
    BE = 512
    assert E % BE == 0

    def body(ea_ref, idx_ref):
        i = pl.program_id(0)
        ea = ea_ref[...]
        e = i * BE + lax.broadcasted_iota(jnp.int32, (BE,), 0)
        idx_ref[...] = (ea[:, 0] * (n1 * n2) + ea[:, 1] * n2 + ea[:, 2]
                        + (e // per_w) * CPAD)

    return pl.pallas_call(
        body,
        grid=(E // BE,),
        in_specs=[pl.BlockSpec((BE, 3), lambda i: (i, 0))],
        out_specs=pl.BlockSpec((BE,), lambda i: (i,)),
        out_shape=jax.ShapeDtypeStruct((E,), jnp.int32),
    )(edge_attr)


def _sc_lookup(table, idx, E):
    info = plsc.get_sparse_core_info()
    NC, NS = info.num_cores, info.num_subcores
    NW = NC * NS
    per_w = E // NW
    CH = 80
    n_ch = per_w // CH
    assert per_w % CH == 0 and E % NW == 0

    NBUF = 5
    LAG = 2
    assert n_ch % NBUF == 0 and n_ch >= 2 * NBUF

    mesh = plsc.VectorSubcoreMesh(core_axis_name="c", subcore_axis_name="s")

    @functools.partial(
        pl.kernel,
        mesh=mesh,
        out_type=jax.ShapeDtypeStruct((E, D), jnp.float32),
        scratch_types=[
            pltpu.VMEM((per_w,), jnp.int32),
            pltpu.VMEM((NBUF * CH, D), jnp.float32),
            *([pltpu.SemaphoreType.DMA] * (2 * NBUF)),
        ],
    )
    def k(t_hbm, idx_hbm, out_hbm, idx_v, rows_v, *sems):
        sem_g, sem_s = sems[:NBUF], sems[NBUF:]
        wid = lax.axis_index("s") * NC + lax.axis_index("c")
        base = wid * per_w
        pltpu.sync_copy(idx_hbm.at[pl.ds(base, per_w)], idx_v)

        def buf(b):
            return rows_v.at[pl.ds(b * CH, CH)]

        def start_gather(kk, b):
            pltpu.async_copy(t_hbm.at[idx_v.at[pl.ds(kk * CH, CH)]],
                             buf(b), sem_g[b])

        def start_scatter(kk, b):
            pltpu.async_copy(buf(b), out_hbm.at[pl.ds(base + kk * CH, CH)],
                             sem_s[b])

        def wait_gather(b):
            pltpu.make_async_copy(out_hbm.at[pl.ds(base, CH)], buf(b),
                                  sem_g[b]).wait()

        def wait_scatter(b):
            pltpu.make_async_copy(buf(b), out_hbm.at[pl.ds(base, CH)],
                                  sem_s[b]).wait()

        for kb in range(NBUF):
            start_gather(kb, kb)
            if kb >= LAG:
                wait_gather(kb - LAG)
                start_scatter(kb - LAG, kb - LAG)

        def outer(g, carry):
            k0 = NBUF + g * NBUF
            for b in range(NBUF):
                kk = k0 + b
                wait_scatter(b)
                start_gather(kk, b)
                bs = (b - LAG) % NBUF
                wait_gather(bs)
                start_scatter(kk - LAG, bs)
            return carry

        lax.fori_loop(0, (n_ch - NBUF) // NBUF, outer, 0)

        for i in range(LAG):
            kk = n_ch - LAG + i
            wait_gather(kk % NBUF)
            start_scatter(kk, kk % NBUF)
        for b in range(NBUF):
            wait_scatter(b)

    return k(table, idx)


def kernel(edge_attr, W0, W1, W2):
    E = edge_attr.shape[0]
    n0, n1, n2 = W0.shape[0], W1.shape[0], W2.shape[0]
    assert n0 * n1 * n2 <= CPAD and max(n0, n1, n2) <= KPAD

    def pad(w):
        return jnp.pad(w, ((0, KPAD - w.shape[0]), (0, 0)))

    REP = 32
    per_w = E // REP
    table = _fused_table(pad(W0), pad(W1), pad(W2), n1, n2, REP)
    idx = _combined_index(edge_attr, n1, n2, per_w)
    return _sc_lookup(table.reshape(REP * CPAD, D), idx, E)


# speedup vs baseline: 7.5103x; 2.8365x over previous
import functools

import jax
import jax.numpy as jnp
from jax import lax
from jax.experimental import pallas as pl
from jax.experimental.pallas import tpu as pltpu
from jax.experimental.pallas import tpu_sc as plsc

D = 128
CPAD = 128
KPAD = 8


def _fused_table(w0p, w1p, w2p, n1, n2, rep):
    def body(w0_ref, w1_ref, w2_ref, t_ref):
        ci = lax.broadcasted_iota(jnp.int32, (CPAD, KPAD), 0)
        j = lax.broadcasted_iota(jnp.int32, (CPAD, KPAD), 1)
        oh0 = (ci // (n1 * n2) == j).astype(jnp.float32)
        oh1 = ((ci // n2) % n1 == j).astype(jnp.float32)
        oh2 = (ci % n2 == j).astype(jnp.float32)
        t = jnp.dot(oh0, w0_ref[...], preferred_element_type=jnp.float32)
        t += jnp.dot(oh1, w1_ref[...], preferred_element_type=jnp.float32)
        t += jnp.dot(oh2, w2_ref[...], preferred_element_type=jnp.float32)
        t_ref[...] = t

    return pl.pallas_call(
        body,
        grid=(rep,),
        in_specs=[pl.BlockSpec((KPAD, D), lambda i: (0, 0))] * 3,
        out_specs=pl.BlockSpec((CPAD, D), lambda i: (i, 0)),
        out_shape=jax.ShapeDtypeStruct((rep * CPAD, D), jnp.float32),
    )(w0p, w1p, w2p)


def _combined_index(ea_t, n1, n2, per_w):
    _, E = ea_t.shape

    def body(ea_ref, idx_ref):
        a, b, c = ea_ref[0], ea_ref[1], ea_ref[2]
        e = lax.broadcasted_iota(jnp.int32, (E,), 0)
        idx_ref[...] = (a * (n1 * n2) + b * n2 + c + (e // per_w) * CPAD)

    return pl.pallas_call(
        body,
        out_shape=jax.ShapeDtypeStruct((E,), jnp.int32),
    )(ea_t)


def _sc_lookup(table, idx, E):
    info = plsc.get_sparse_core_info()
    NC, NS = info.num_cores, info.num_subcores
    NW = NC * NS
    per_w = E // NW
    CH = 80
    n_ch = per_w // CH
    assert per_w % CH == 0 and E % NW == 0

    NBUF = 5
    LAG = 2
    assert n_ch % NBUF == 0 and n_ch >= 2 * NBUF

    mesh = plsc.VectorSubcoreMesh(core_axis_name="c", subcore_axis_name="s")

    @functools.partial(
        pl.kernel,
        mesh=mesh,
        out_type=jax.ShapeDtypeStruct((E, D), jnp.float32),
        scratch_types=[
            pltpu.VMEM((per_w,), jnp.int32),
            pltpu.VMEM((NBUF * CH, D), jnp.float32),
            *([pltpu.SemaphoreType.DMA] * (2 * NBUF)),
        ],
    )
    def k(t_hbm, idx_hbm, out_hbm, idx_v, rows_v, *sems):
        sem_g, sem_s = sems[:NBUF], sems[NBUF:]
        wid = lax.axis_index("s") * NC + lax.axis_index("c")
        base = wid * per_w
        pltpu.sync_copy(idx_hbm.at[pl.ds(base, per_w)], idx_v)

        def buf(b):
            return rows_v.at[pl.ds(b * CH, CH)]

        def start_gather(kk, b):
            pltpu.async_copy(t_hbm.at[idx_v.at[pl.ds(kk * CH, CH)]],
                             buf(b), sem_g[b])

        def start_scatter(kk, b):
            pltpu.async_copy(buf(b), out_hbm.at[pl.ds(base + kk * CH, CH)],
                             sem_s[b])

        def wait_gather(b):
            pltpu.make_async_copy(out_hbm.at[pl.ds(base, CH)], buf(b),
                                  sem_g[b]).wait()

        def wait_scatter(b):
            pltpu.make_async_copy(buf(b), out_hbm.at[pl.ds(base, CH)],
                                  sem_s[b]).wait()

        for kb in range(NBUF):
            start_gather(kb, kb)
            if kb >= LAG:
                wait_gather(kb - LAG)
                start_scatter(kb - LAG, kb - LAG)

        def outer(g, carry):
            k0 = NBUF + g * NBUF
            for b in range(NBUF):
                kk = k0 + b
                wait_scatter(b)
                start_gather(kk, b)
                bs = (b - LAG) % NBUF
                wait_gather(bs)
                start_scatter(kk - LAG, bs)
            return carry

        lax.fori_loop(0, (n_ch - NBUF) // NBUF, outer, 0)

        for i in range(LAG):
            kk = n_ch - LAG + i
            wait_gather(kk % NBUF)
            start_scatter(kk, kk % NBUF)
        for b in range(NBUF):
            wait_scatter(b)

    return k(table, idx)


def kernel(edge_attr, W0, W1, W2):
    E = edge_attr.shape[0]
    n0, n1, n2 = W0.shape[0], W1.shape[0], W2.shape[0]
    assert n0 * n1 * n2 <= CPAD and max(n0, n1, n2) <= KPAD

    def pad(w):
        return jnp.pad(w, ((0, KPAD - w.shape[0]), (0, 0)))

    REP = 32
    per_w = E // REP
    table = _fused_table(pad(W0), pad(W1), pad(W2), n1, n2, REP)
    ea_t = jnp.transpose(edge_attr)
    idx = _combined_index(ea_t, n1, n2, per_w)
    return _sc_lookup(table, idx, E)


# speedup vs baseline: 20.7439x; 2.7621x over previous
import functools

import jax
import jax.numpy as jnp
from jax import lax
from jax.experimental import pallas as pl
from jax.experimental.pallas import tpu as pltpu
from jax.experimental.pallas import tpu_sc as plsc

D = 128
CPAD = 128
KPAD = 8


def _fused_table(w0p, w1p, w2p, n1, n2, rep):
    def body(w0_ref, w1_ref, w2_ref, t_ref):
        ci = lax.broadcasted_iota(jnp.int32, (CPAD, KPAD), 0)
        j = lax.broadcasted_iota(jnp.int32, (CPAD, KPAD), 1)
        oh0 = (ci // (n1 * n2) == j).astype(jnp.float32)
        oh1 = ((ci // n2) % n1 == j).astype(jnp.float32)
        oh2 = (ci % n2 == j).astype(jnp.float32)
        t = jnp.dot(oh0, w0_ref[...], preferred_element_type=jnp.float32)
        t += jnp.dot(oh1, w1_ref[...], preferred_element_type=jnp.float32)
        t += jnp.dot(oh2, w2_ref[...], preferred_element_type=jnp.float32)
        t_ref[...] = t

    return pl.pallas_call(
        body,
        grid=(rep,),
        in_specs=[pl.BlockSpec((KPAD, D), lambda i: (0, 0))] * 3,
        out_specs=pl.BlockSpec((CPAD, D), lambda i: (i, 0)),
        out_shape=jax.ShapeDtypeStruct((rep * CPAD, D), jnp.float32),
    )(w0p, w1p, w2p)


def _combined_index(ea_t, n1, n2, per_w):
    _, E = ea_t.shape

    def body(ea_ref, idx_ref):
        a, b, c = ea_ref[0], ea_ref[1], ea_ref[2]
        e = lax.broadcasted_iota(jnp.int32, (E,), 0)
        idx_ref[...] = (a * (n1 * n2) + b * n2 + c + (e // (2 * per_w)) * CPAD)

    return pl.pallas_call(
        body,
        out_shape=jax.ShapeDtypeStruct((E,), jnp.int32),
    )(ea_t)


def _sc_lookup(table, idx, E):
    info = plsc.get_sparse_core_info()
    NC, NS = info.num_cores, info.num_subcores
    NW = NC * NS
    per_w = E // NW
    CH = 80
    n_ch = per_w // CH
    assert per_w % CH == 0 and E % NW == 0

    NBUF = 5
    LAG = 2
    assert n_ch % NBUF == 0 and n_ch >= 2 * NBUF

    mesh = plsc.VectorSubcoreMesh(core_axis_name="c", subcore_axis_name="s")

    @functools.partial(
        pl.kernel,
        mesh=mesh,
        out_type=jax.ShapeDtypeStruct((E, D), jnp.float32),
        scratch_types=[
            pltpu.VMEM_SHARED((NS * CPAD, D), jnp.float32),
            pltpu.VMEM((per_w,), jnp.int32),
            pltpu.VMEM((NBUF * CH, D), jnp.float32),
            *([pltpu.SemaphoreType.DMA] * (2 * NBUF)),
        ],
    )
    def k(t_hbm, idx_hbm, out_hbm, t_sh, idx_v, rows_v, *sems):
        sem_g, sem_s = sems[:NBUF], sems[NBUF:]
        s = lax.axis_index("s")
        wid = s * NC + lax.axis_index("c")
        base = wid * per_w
        pltpu.sync_copy(t_hbm.at[pl.ds(s * CPAD, CPAD)],
                        t_sh.at[pl.ds(s * CPAD, CPAD)])
        pltpu.sync_copy(idx_hbm.at[pl.ds(base, per_w)], idx_v)

        def buf(b):
            return rows_v.at[pl.ds(b * CH, CH)]

        def start_gather(kk, b):
            pltpu.async_copy(t_sh.at[idx_v.at[pl.ds(kk * CH, CH)]],
                             buf(b), sem_g[b])

        def start_scatter(kk, b):
            pltpu.async_copy(buf(b), out_hbm.at[pl.ds(base + kk * CH, CH)],
                             sem_s[b])

        def wait_gather(b):
            pltpu.make_async_copy(out_hbm.at[pl.ds(base, CH)], buf(b),
                                  sem_g[b]).wait()

        def wait_scatter(b):
            pltpu.make_async_copy(buf(b), out_hbm.at[pl.ds(base, CH)],
                                  sem_s[b]).wait()

        for kb in range(NBUF):
            start_gather(kb, kb)
            if kb >= LAG:
                wait_gather(kb - LAG)
                start_scatter(kb - LAG, kb - LAG)

        def outer(g, carry):
            k0 = NBUF + g * NBUF
            for b in range(NBUF):
                kk = k0 + b
                wait_scatter(b)
                start_gather(kk, b)
                bs = (b - LAG) % NBUF
                wait_gather(bs)
                start_scatter(kk - LAG, bs)
            return carry

        lax.fori_loop(0, (n_ch - NBUF) // NBUF, outer, 0)

        for i in range(LAG):
            kk = n_ch - LAG + i
            wait_gather(kk % NBUF)
            start_scatter(kk, kk % NBUF)
        for b in range(NBUF):
            wait_scatter(b)

    return k(table, idx)


def kernel(edge_attr, W0, W1, W2):
    E = edge_attr.shape[0]
    n0, n1, n2 = W0.shape[0], W1.shape[0], W2.shape[0]
    assert n0 * n1 * n2 <= CPAD and max(n0, n1, n2) <= KPAD

    def pad(w):
        return jnp.pad(w, ((0, KPAD - w.shape[0]), (0, 0)))

    REP = 16
    per_w = E // 32
    table = _fused_table(pad(W0), pad(W1), pad(W2), n1, n2, REP)
    ea_t = jnp.transpose(edge_attr)
    idx = _combined_index(ea_t, n1, n2, per_w)
    return _sc_lookup(table, idx, E)


# speedup vs baseline: 22.6860x; 1.0936x over previous
import functools

import jax
import jax.numpy as jnp
from jax import lax
from jax.experimental import pallas as pl
from jax.experimental.pallas import tpu as pltpu
from jax.experimental.pallas import tpu_sc as plsc

D = 128
CPAD = 128
REP_HBM = 4


def _fused_table(w0p, w1p, w2p, n1, n2, rep):
    n0 = w0p.shape[0]

    def body(w0_ref, w1_ref, w2_ref, t_ref):
        def oh(vals, n):
            j = lax.broadcasted_iota(jnp.int32, (CPAD, n), 1)
            return (vals == j).astype(jnp.float32)

        ci = lax.broadcasted_iota(jnp.int32, (CPAD, 1), 0)
        t = jnp.dot(oh(ci // (n1 * n2), n0), w0_ref[...],
                    preferred_element_type=jnp.float32)
        t += jnp.dot(oh((ci // n2) % n1, n1), w1_ref[...],
                     preferred_element_type=jnp.float32)
        t += jnp.dot(oh(ci % n2, n2), w2_ref[...],
                     preferred_element_type=jnp.float32)
        t_ref[...] = t

    return pl.pallas_call(
        body,
        grid=(rep,),
        in_specs=[pl.BlockSpec(w.shape, lambda i: (0, 0))
                  for w in (w0p, w1p, w2p)],
        out_specs=pl.BlockSpec((CPAD, D), lambda i: (i, 0)),
        out_shape=jax.ShapeDtypeStruct((rep * CPAD, D), jnp.float32),
    )(w0p, w1p, w2p)


def _combined_index(ea_t, n1, n2, per_w):
    _, E = ea_t.shape

    def body(ea_ref, idx_ref):
        a, b, c = ea_ref[0], ea_ref[1], ea_ref[2]
        e = lax.broadcasted_iota(jnp.int32, (E,), 0)
        idx_ref[...] = (a * (n1 * n2) + b * n2 + c + (e // (2 * per_w)) * CPAD)

    return pl.pallas_call(
        body,
        out_shape=jax.ShapeDtypeStruct((E,), jnp.int32),
    )(ea_t)


def _sc_lookup(table, idx, E):
    info = plsc.get_sparse_core_info()
    NC, NS = info.num_cores, info.num_subcores
    NW = NC * NS
    per_w = E // NW
    CH = 80
    n_ch = per_w // CH
    assert per_w % CH == 0 and E % NW == 0

    NBUF = 5
    LAG = 2
    assert n_ch % NBUF == 0 and n_ch >= 2 * NBUF

    mesh = plsc.VectorSubcoreMesh(core_axis_name="c", subcore_axis_name="s")

    @functools.partial(
        pl.kernel,
        mesh=mesh,
        out_type=jax.ShapeDtypeStruct((E, D), jnp.float32),
        scratch_types=[
            pltpu.VMEM_SHARED((NS * CPAD, D), jnp.float32),
            pltpu.VMEM((per_w,), jnp.int32),
            pltpu.VMEM((NBUF * CH, D), jnp.float32),
            *([pltpu.SemaphoreType.DMA] * (2 * NBUF)),
        ],
    )
    def k(t_hbm, idx_hbm, out_hbm, t_sh, idx_v, rows_v, *sems):
        sem_g, sem_s = sems[:NBUF], sems[NBUF:]
        s = lax.axis_index("s")
        wid = s * NC + lax.axis_index("c")
        base = wid * per_w
        pltpu.sync_copy(t_hbm.at[pl.ds((s % REP_HBM) * CPAD, CPAD)],
                        t_sh.at[pl.ds(s * CPAD, CPAD)])
        pltpu.sync_copy(idx_hbm.at[pl.ds(base, per_w)], idx_v)

        def buf(b):
            return rows_v.at[pl.ds(b * CH, CH)]

        def start_gather(kk, b):
            pltpu.async_copy(t_sh.at[idx_v.at[pl.ds(kk * CH, CH)]],
                             buf(b), sem_g[b])

        def start_scatter(kk, b):
            pltpu.async_copy(buf(b), out_hbm.at[pl.ds(base + kk * CH, CH)],
                             sem_s[b])

        def wait_gather(b):
            pltpu.make_async_copy(out_hbm.at[pl.ds(base, CH)], buf(b),
                                  sem_g[b]).wait()

        def wait_scatter(b):
            pltpu.make_async_copy(buf(b), out_hbm.at[pl.ds(base, CH)],
                                  sem_s[b]).wait()

        for kb in range(NBUF):
            start_gather(kb, kb)
            if kb >= LAG:
                wait_gather(kb - LAG)
                start_scatter(kb - LAG, kb - LAG)

        def outer(g, carry):
            k0 = NBUF + g * NBUF
            for b in range(NBUF):
                kk = k0 + b
                wait_scatter(b)
                start_gather(kk, b)
                bs = (b - LAG) % NBUF
                wait_gather(bs)
                start_scatter(kk - LAG, bs)
            return carry

        lax.fori_loop(0, (n_ch - NBUF) // NBUF, outer, 0)

        for i in range(LAG):
            kk = n_ch - LAG + i
            wait_gather(kk % NBUF)
            start_scatter(kk, kk % NBUF)
        for b in range(NBUF):
            wait_scatter(b)

    return k(table, idx)


def kernel(edge_attr, W0, W1, W2):
    E = edge_attr.shape[0]
    n0, n1, n2 = W0.shape[0], W1.shape[0], W2.shape[0]
    assert n0 * n1 * n2 <= CPAD

    per_w = E // 32
    table = _fused_table(W0, W1, W2, n1, n2, REP_HBM)
    ea_t = jnp.transpose(edge_attr)
    idx = _combined_index(ea_t, n1, n2, per_w)
    return _sc_lookup(table, idx, E)


# speedup vs baseline: 22.6994x; 1.0006x over previous
import functools

import jax
import jax.numpy as jnp
from jax import lax
from jax.experimental import pallas as pl
from jax.experimental.pallas import tpu as pltpu
from jax.experimental.pallas import tpu_sc as plsc

D = 128
CPAD = 128
REP_HBM = 4


def _fused_table(w0p, w1p, w2p, n1, n2, rep):
    n0 = w0p.shape[0]

    def body(w0_ref, w1_ref, w2_ref, t_ref):
        def oh(vals, n):
            j = lax.broadcasted_iota(jnp.int32, (CPAD, n), 1)
            return (vals == j).astype(jnp.float32)

        ci = lax.broadcasted_iota(jnp.int32, (CPAD, 1), 0)
        t = jnp.dot(oh(ci // (n1 * n2), n0), w0_ref[...],
                    preferred_element_type=jnp.float32)
        t += jnp.dot(oh((ci // n2) % n1, n1), w1_ref[...],
                     preferred_element_type=jnp.float32)
        t += jnp.dot(oh(ci % n2, n2), w2_ref[...],
                     preferred_element_type=jnp.float32)
        t_ref[...] = t

    return pl.pallas_call(
        body,
        grid=(rep,),
        in_specs=[pl.BlockSpec(w.shape, lambda i: (0, 0))
                  for w in (w0p, w1p, w2p)],
        out_specs=pl.BlockSpec((CPAD, D), lambda i: (i, 0)),
        out_shape=jax.ShapeDtypeStruct((rep * CPAD, D), jnp.float32),
    )(w0p, w1p, w2p)


def _combined_index(ea_t, n1, n2, per_w):
    _, E = ea_t.shape

    def body(ea_ref, idx_ref):
        a, b, c = ea_ref[0], ea_ref[1], ea_ref[2]
        e = lax.broadcasted_iota(jnp.int32, (E,), 0)
        idx_ref[...] = (a * (n1 * n2) + b * n2 + c + (e // (2 * per_w)) * CPAD)

    return pl.pallas_call(
        body,
        out_shape=jax.ShapeDtypeStruct((E,), jnp.int32),
    )(ea_t)


def _sc_lookup(table, idx, E):
    info = plsc.get_sparse_core_info()
    NC, NS = info.num_cores, info.num_subcores
    NW = NC * NS
    per_w = E // NW
    CH = 80
    n_ch = per_w // CH
    assert per_w % CH == 0 and E % NW == 0

    NBUF = 5
    LAG = 3
    assert n_ch % NBUF == 0 and n_ch >= 2 * NBUF

    mesh = plsc.VectorSubcoreMesh(core_axis_name="c", subcore_axis_name="s")

    @functools.partial(
        pl.kernel,
        mesh=mesh,
        out_type=jax.ShapeDtypeStruct((E, D), jnp.float32),
        scratch_types=[
            pltpu.VMEM_SHARED((NS * CPAD, D), jnp.float32),
            pltpu.VMEM((per_w,), jnp.int32),
            pltpu.VMEM((NBUF * CH, D), jnp.float32),
            *([pltpu.SemaphoreType.DMA] * (2 * NBUF)),
        ],
    )
    def k(t_hbm, idx_hbm, out_hbm, t_sh, idx_v, rows_v, *sems):
        sem_g, sem_s = sems[:NBUF], sems[NBUF:]
        s = lax.axis_index("s")
        wid = s * NC + lax.axis_index("c")
        base = wid * per_w
        pltpu.sync_copy(t_hbm.at[pl.ds((s % REP_HBM) * CPAD, CPAD)],
                        t_sh.at[pl.ds(s * CPAD, CPAD)])
        pltpu.sync_copy(idx_hbm.at[pl.ds(base, per_w)], idx_v)

        def buf(b):
            return rows_v.at[pl.ds(b * CH, CH)]

        def start_gather(kk, b):
            pltpu.async_copy(t_sh.at[idx_v.at[pl.ds(kk * CH, CH)]],
                             buf(b), sem_g[b])

        def start_scatter(kk, b):
            pltpu.async_copy(buf(b), out_hbm.at[pl.ds(base + kk * CH, CH)],
                             sem_s[b])

        def wait_gather(b):
            pltpu.make_async_copy(out_hbm.at[pl.ds(base, CH)], buf(b),
                                  sem_g[b]).wait()

        def wait_scatter(b):
            pltpu.make_async_copy(buf(b), out_hbm.at[pl.ds(base, CH)],
                                  sem_s[b]).wait()

        for kb in range(NBUF):
            start_gather(kb, kb)
            if kb >= LAG:
                wait_gather(kb - LAG)
                start_scatter(kb - LAG, kb - LAG)

        def outer(g, carry):
            k0 = NBUF + g * NBUF
            for b in range(NBUF):
                kk = k0 + b
                wait_scatter(b)
                start_gather(kk, b)
                bs = (b - LAG) % NBUF
                wait_gather(bs)
                start_scatter(kk - LAG, bs)
            return carry

        lax.fori_loop(0, (n_ch - NBUF) // NBUF, outer, 0)

        for i in range(LAG):
            kk = n_ch - LAG + i
            wait_gather(kk % NBUF)
            start_scatter(kk, kk % NBUF)
        for b in range(NBUF):
            wait_scatter(b)

    return k(table, idx)


def kernel(edge_attr, W0, W1, W2):
    E = edge_attr.shape[0]
    n0, n1, n2 = W0.shape[0], W1.shape[0], W2.shape[0]
    assert n0 * n1 * n2 <= CPAD

    per_w = E // 32
    table = _fused_table(W0, W1, W2, n1, n2, REP_HBM)
    ea_t = jnp.transpose(edge_attr)
    idx = _combined_index(ea_t, n1, n2, per_w)
    return _sc_lookup(table, idx, E)


# speedup vs baseline: 22.9342x; 1.0103x over previous
import functools

import jax
import jax.numpy as jnp
from jax import lax
from jax.experimental import pallas as pl
from jax.experimental.pallas import tpu as pltpu
from jax.experimental.pallas import tpu_sc as plsc

D = 128
CPAD = 128
REP_HBM = 4


def _tc_prelude(ea_t, w0, w1, w2, n1, n2, per_w):
    _, E = ea_t.shape
    n0 = w0.shape[0]

    def body(ea_ref, w0_ref, w1_ref, w2_ref, t_ref, idx_ref):
        def oh(vals, n):
            j = lax.broadcasted_iota(jnp.int32, (CPAD, n), 1)
            return (vals == j).astype(jnp.float32)

        ci = lax.broadcasted_iota(jnp.int32, (CPAD, 1), 0)
        t = jnp.dot(oh(ci // (n1 * n2), n0), w0_ref[...],
                    preferred_element_type=jnp.float32)
        t += jnp.dot(oh((ci // n2) % n1, n1), w1_ref[...],
                     preferred_element_type=jnp.float32)
        t += jnp.dot(oh(ci % n2, n2), w2_ref[...],
                     preferred_element_type=jnp.float32)
        for r in range(REP_HBM):
            t_ref[pl.ds(r * CPAD, CPAD)] = t

        a, b, c = ea_ref[0], ea_ref[1], ea_ref[2]
        e = lax.broadcasted_iota(jnp.int32, (E,), 0)
        idx_ref[...] = (a * (n1 * n2) + b * n2 + c + (e // (2 * per_w)) * CPAD)

    return pl.pallas_call(
        body,
        out_shape=(
            jax.ShapeDtypeStruct((REP_HBM * CPAD, D), jnp.float32),
            jax.ShapeDtypeStruct((E,), jnp.int32),
        ),
    )(ea_t, w0, w1, w2)


def _sc_lookup(table, idx, E):
    info = plsc.get_sparse_core_info()
    NC, NS = info.num_cores, info.num_subcores
    NW = NC * NS
    per_w = E // NW
    CH = 80
    n_ch = per_w // CH
    assert per_w % CH == 0 and E % NW == 0

    NBUF = 5
    LAG = 3
    assert n_ch % NBUF == 0 and n_ch >= 2 * NBUF

    mesh = plsc.VectorSubcoreMesh(core_axis_name="c", subcore_axis_name="s")

    @functools.partial(
        pl.kernel,
        mesh=mesh,
        out_type=jax.ShapeDtypeStruct((E, D), jnp.float32),
        scratch_types=[
            pltpu.VMEM_SHARED((NS * CPAD, D), jnp.float32),
            pltpu.VMEM((per_w,), jnp.int32),
            pltpu.VMEM((NBUF * CH, D), jnp.float32),
            *([pltpu.SemaphoreType.DMA] * (2 * NBUF)),
        ],
    )
    def k(t_hbm, idx_hbm, out_hbm, t_sh, idx_v, rows_v, *sems):
        sem_g, sem_s = sems[:NBUF], sems[NBUF:]
        s = lax.axis_index("s")
        wid = s * NC + lax.axis_index("c")
        base = wid * per_w
        pltpu.sync_copy(t_hbm.at[pl.ds((s % REP_HBM) * CPAD, CPAD)],
                        t_sh.at[pl.ds(s * CPAD, CPAD)])
        pltpu.sync_copy(idx_hbm.at[pl.ds(base, per_w)], idx_v)

        def buf(b):
            return rows_v.at[pl.ds(b * CH, CH)]

        def start_gather(kk, b):
            pltpu.async_copy(t_sh.at[idx_v.at[pl.ds(kk * CH, CH)]],
                             buf(b), sem_g[b])

        def start_scatter(kk, b):
            pltpu.async_copy(buf(b), out_hbm.at[pl.ds(base + kk * CH, CH)],
                             sem_s[b])

        def wait_gather(b):
            pltpu.make_async_copy(out_hbm.at[pl.ds(base, CH)], buf(b),
                                  sem_g[b]).wait()

        def wait_scatter(b):
            pltpu.make_async_copy(buf(b), out_hbm.at[pl.ds(base, CH)],
                                  sem_s[b]).wait()

        for kb in range(NBUF):
            start_gather(kb, kb)
            if kb >= LAG:
                wait_gather(kb - LAG)
                start_scatter(kb - LAG, kb - LAG)

        def outer(g, carry):
            k0 = NBUF + g * NBUF
            for b in range(NBUF):
                kk = k0 + b
                wait_scatter(b)
                start_gather(kk, b)
                bs = (b - LAG) % NBUF
                wait_gather(bs)
                start_scatter(kk - LAG, bs)
            return carry

        lax.fori_loop(0, (n_ch - NBUF) // NBUF, outer, 0)

        for i in range(LAG):
            kk = n_ch - LAG + i
            wait_gather(kk % NBUF)
            start_scatter(kk, kk % NBUF)
        for b in range(NBUF):
            wait_scatter(b)

    return k(table, idx)


def kernel(edge_attr, W0, W1, W2):
    E = edge_attr.shape[0]
    n0, n1, n2 = W0.shape[0], W1.shape[0], W2.shape[0]
    assert n0 * n1 * n2 <= CPAD

    per_w = E // 32
    ea_t = jnp.transpose(edge_attr)
    table, idx = _tc_prelude(ea_t, W0, W1, W2, n1, n2, per_w)
    return _sc_lookup(table, idx, E)


# speedup vs baseline: 23.1407x; 1.0090x over previous
import functools

import jax
import jax.numpy as jnp
from jax import lax
from jax.experimental import pallas as pl
from jax.experimental.pallas import tpu as pltpu
from jax.experimental.pallas import tpu_sc as plsc

D = 128
CPAD = 128
REP_HBM = 4


def _tc_prelude(ea_t, w0, w1, w2, n1, n2, per_w):
    _, E = ea_t.shape
    n0 = w0.shape[0]

    def body(ea_ref, w0_ref, w1_ref, w2_ref, t_ref, idx_ref):
        def oh(vals, n):
            j = lax.broadcasted_iota(jnp.int32, (CPAD, n), 1)
            return (vals == j).astype(jnp.float32)

        ci = lax.broadcasted_iota(jnp.int32, (CPAD, 1), 0)
        t = jnp.dot(oh(ci // (n1 * n2), n0), w0_ref[...],
                    preferred_element_type=jnp.float32)
        t += jnp.dot(oh((ci // n2) % n1, n1), w1_ref[...],
                     preferred_element_type=jnp.float32)
        t += jnp.dot(oh(ci % n2, n2), w2_ref[...],
                     preferred_element_type=jnp.float32)
        for r in range(REP_HBM):
            t_ref[pl.ds(r * CPAD, CPAD)] = t

        a, b, c = ea_ref[0], ea_ref[1], ea_ref[2]
        e = lax.broadcasted_iota(jnp.int32, (E,), 0)
        idx_ref[...] = (a * (n1 * n2) + b * n2 + c + (e // (2 * per_w)) * CPAD)

    return pl.pallas_call(
        body,
        out_shape=(
            jax.ShapeDtypeStruct((REP_HBM * CPAD, D), jnp.float32),
            jax.ShapeDtypeStruct((E,), jnp.int32),
        ),
    )(ea_t, w0, w1, w2)


def _sc_lookup(table, idx, E):
    info = plsc.get_sparse_core_info()
    NC, NS = info.num_cores, info.num_subcores
    NW = NC * NS
    per_w = E // NW
    CH = 80
    n_ch = per_w // CH
    assert per_w % CH == 0 and E % NW == 0

    NBUF = 5
    LAG = 3
    assert n_ch % NBUF == 0 and n_ch >= 2 * NBUF

    mesh = plsc.VectorSubcoreMesh(core_axis_name="c", subcore_axis_name="s")

    @functools.partial(
        pl.kernel,
        mesh=mesh,
        out_type=jax.ShapeDtypeStruct((E, D), jnp.float32),
        scratch_types=[
            pltpu.VMEM_SHARED((NS * CPAD, D), jnp.float32),
            pltpu.VMEM((per_w,), jnp.int32),
            pltpu.VMEM((NBUF * CH, D), jnp.float32),
            *([pltpu.SemaphoreType.DMA] * (2 * NBUF)),
        ],
    )
    def k(t_hbm, idx_hbm, out_hbm, t_sh, idx_v, rows_v, *sems):
        sem_g, sem_s = sems[:NBUF], sems[NBUF:]
        s = lax.axis_index("s")
        wid = s * NC + lax.axis_index("c")
        base = wid * per_w
        tcopy = pltpu.async_copy(t_hbm.at[pl.ds((s % REP_HBM) * CPAD, CPAD)],
                                 t_sh.at[pl.ds(s * CPAD, CPAD)], sem_g[0])
        icopy = pltpu.async_copy(idx_hbm.at[pl.ds(base, per_w)], idx_v,
                                 sem_g[1])
        tcopy.wait()
        icopy.wait()

        def buf(b):
            return rows_v.at[pl.ds(b * CH, CH)]

        def start_gather(kk, b):
            pltpu.async_copy(t_sh.at[idx_v.at[pl.ds(kk * CH, CH)]],
                             buf(b), sem_g[b])

        def start_scatter(kk, b):
            pltpu.async_copy(buf(b), out_hbm.at[pl.ds(base + kk * CH, CH)],
                             sem_s[b])

        def wait_gather(b):
            pltpu.make_async_copy(out_hbm.at[pl.ds(base, CH)], buf(b),
                                  sem_g[b]).wait()

        def wait_scatter(b):
            pltpu.make_async_copy(buf(b), out_hbm.at[pl.ds(base, CH)],
                                  sem_s[b]).wait()

        for kb in range(NBUF):
            start_gather(kb, kb)
            if kb >= LAG:
                wait_gather(kb - LAG)
                start_scatter(kb - LAG, kb - LAG)

        def outer(g, carry):
            k0 = NBUF + g * NBUF
            for b in range(NBUF):
                kk = k0 + b
                wait_scatter(b)
                start_gather(kk, b)
                bs = (b - LAG) % NBUF
                wait_gather(bs)
                start_scatter(kk - LAG, bs)
            return carry

        lax.fori_loop(0, (n_ch - NBUF) // NBUF, outer, 0)

        for i in range(LAG):
            kk = n_ch - LAG + i
            wait_gather(kk % NBUF)
            start_scatter(kk, kk % NBUF)
        for b in range(NBUF):
            wait_scatter(b)

    return k(table, idx)


def kernel(edge_attr, W0, W1, W2):
    E = edge_attr.shape[0]
    n0, n1, n2 = W0.shape[0], W1.shape[0], W2.shape[0]
    assert n0 * n1 * n2 <= CPAD

    per_w = E // 32
    ea_t = jnp.transpose(edge_attr)
    table, idx = _tc_prelude(ea_t, W0, W1, W2, n1, n2, per_w)
    return _sc_lookup(table, idx, E)


# speedup vs baseline: 23.5811x; 1.0190x over previous
import functools

import jax
import jax.numpy as jnp
from jax import lax
from jax.experimental import pallas as pl
from jax.experimental.pallas import tpu as pltpu
from jax.experimental.pallas import tpu_sc as plsc

D = 128
CPAD = 128
REP_HBM = 4


def _tc_prelude(ea_t, w0, w1, w2, n1, n2, per_w):
    _, E = ea_t.shape
    n0 = w0.shape[0]

    def body(ea_ref, w0_ref, w1_ref, w2_ref, t_ref, idx_ref):
        def rows(w_ref, vals):
            acc = jnp.zeros((CPAD, D), jnp.float32)
            for j in range(w_ref.shape[0]):
                acc = jnp.where(vals == j, w_ref[j][None, :], acc)
            return acc

        ci = lax.broadcasted_iota(jnp.int32, (CPAD, 1), 0)
        t = (rows(w0_ref, ci // (n1 * n2))
             + rows(w1_ref, (ci // n2) % n1)
             + rows(w2_ref, ci % n2))
        for r in range(REP_HBM):
            t_ref[pl.ds(r * CPAD, CPAD)] = t

        a, b, c = ea_ref[0], ea_ref[1], ea_ref[2]
        e = lax.broadcasted_iota(jnp.int32, (E,), 0)
        idx_ref[...] = (a * (n1 * n2) + b * n2 + c + (e // (2 * per_w)) * CPAD)

    return pl.pallas_call(
        body,
        out_shape=(
            jax.ShapeDtypeStruct((REP_HBM * CPAD, D), jnp.float32),
            jax.ShapeDtypeStruct((E,), jnp.int32),
        ),
    )(ea_t, w0, w1, w2)


def _sc_lookup(table, idx, E):
    info = plsc.get_sparse_core_info()
    NC, NS = info.num_cores, info.num_subcores
    NW = NC * NS
    per_w = E // NW
    CH = 80
    n_ch = per_w // CH
    assert per_w % CH == 0 and E % NW == 0

    NBUF = 5
    LAG = 3
    assert n_ch % NBUF == 0 and n_ch >= 2 * NBUF

    mesh = plsc.VectorSubcoreMesh(core_axis_name="c", subcore_axis_name="s")

    @functools.partial(
        pl.kernel,
        mesh=mesh,
        out_type=jax.ShapeDtypeStruct((E, D), jnp.float32),
        scratch_types=[
            pltpu.VMEM_SHARED((NS * CPAD, D), jnp.float32),
            pltpu.VMEM((per_w,), jnp.int32),
            pltpu.VMEM((NBUF * CH, D), jnp.float32),
            *([pltpu.SemaphoreType.DMA] * (2 * NBUF)),
        ],
    )
    def k(t_hbm, idx_hbm, out_hbm, t_sh, idx_v, rows_v, *sems):
        sem_g, sem_s = sems[:NBUF], sems[NBUF:]
        s = lax.axis_index("s")
        wid = s * NC + lax.axis_index("c")
        base = wid * per_w
        tcopy = pltpu.async_copy(t_hbm.at[pl.ds((s % REP_HBM) * CPAD, CPAD)],
                                 t_sh.at[pl.ds(s * CPAD, CPAD)], sem_g[0])
        icopy = pltpu.async_copy(idx_hbm.at[pl.ds(base, per_w)], idx_v,
                                 sem_g[1])
        tcopy.wait()
        icopy.wait()

        def buf(b):
            return rows_v.at[pl.ds(b * CH, CH)]

        def start_gather(kk, b):
            pltpu.async_copy(t_sh.at[idx_v.at[pl.ds(kk * CH, CH)]],
                             buf(b), sem_g[b])

        def start_scatter(kk, b):
            pltpu.async_copy(buf(b), out_hbm.at[pl.ds(base + kk * CH, CH)],
                             sem_s[b])

        def wait_gather(b):
            pltpu.make_async_copy(out_hbm.at[pl.ds(base, CH)], buf(b),
                                  sem_g[b]).wait()

        def wait_scatter(b):
            pltpu.make_async_copy(buf(b), out_hbm.at[pl.ds(base, CH)],
                                  sem_s[b]).wait()

        for kb in range(NBUF):
            start_gather(kb, kb)
            if kb >= LAG:
                wait_gather(kb - LAG)
                start_scatter(kb - LAG, kb - LAG)

        def outer(g, carry):
            k0 = NBUF + g * NBUF
            for b in range(NBUF):
                kk = k0 + b
                wait_scatter(b)
                start_gather(kk, b)
                bs = (b - LAG) % NBUF
                wait_gather(bs)
                start_scatter(kk - LAG, bs)
            return carry

        lax.fori_loop(0, (n_ch - NBUF) // NBUF, outer, 0)

        for i in range(LAG):
            kk = n_ch - LAG + i
            wait_gather(kk % NBUF)
            start_scatter(kk, kk % NBUF)
        for b in range(NBUF):
            wait_scatter(b)

    return k(table, idx)


def kernel(edge_attr, W0, W1, W2):
    E = edge_attr.shape[0]
    n0, n1, n2 = W0.shape[0], W1.shape[0], W2.shape[0]
    assert n0 * n1 * n2 <= CPAD

    per_w = E // 32
    ea_t = jnp.transpose(edge_attr)
    table, idx = _tc_prelude(ea_t, W0, W1, W2, n1, n2, per_w)
    return _sc_lookup(table, idx, E)
